# baseline (speedup 1.0000x reference)
import functools

import jax
import jax.numpy as jnp
from jax import lax
from jax.experimental import pallas as pl
from jax.experimental.pallas import tpu as pltpu
from jax.experimental.pallas import tpu_sc as plsc

K_CODES = 1024
CODE_DIM = 64
ROW_BLOCK = 4096
NC = 2
NS = 16
NW = NC * NS


def _argmin_body(x_ref, cb_ref, idx_ref):
    x = x_ref[...]
    cb = cb_ref[...]
    xs = x * (-2.0)
    s2 = jax.lax.dot_general(xs, cb, (((1,), (1,)), ((), ())))
    x2 = jnp.sum(x * x, axis=1, keepdims=True)
    c2 = jnp.sum(cb * cb, axis=1)
    d = (x2 + c2[None, :]) + s2
    dmin = jnp.min(d, axis=1, keepdims=True)
    iota_k = jax.lax.broadcasted_iota(jnp.int32, (ROW_BLOCK, K_CODES), 1)
    idx_ref[...] = jnp.min(jnp.where(d == dmin, iota_k, K_CODES), axis=1)


def _vq_argmin(flat, codebook):
    n = flat.shape[0]
    return pl.pallas_call(
        _argmin_body,
        grid=(n // ROW_BLOCK,),
        in_specs=[
            pl.BlockSpec((ROW_BLOCK, CODE_DIM), lambda i: (i, 0)),
            pl.BlockSpec((K_CODES, CODE_DIM), lambda i: (0, 0)),
        ],
        out_specs=pl.BlockSpec((ROW_BLOCK,), lambda i: (i,)),
        out_shape=jax.ShapeDtypeStruct((n,), jnp.int32),
    )(flat, codebook)


def _make_sc_gather(n):
    b_per_w = n // NW
    mesh = plsc.VectorSubcoreMesh(core_axis_name="c", subcore_axis_name="s")

    @functools.partial(
        pl.kernel,
        mesh=mesh,
        out_type=jax.ShapeDtypeStruct((n, CODE_DIM), jnp.float32),
        scratch_types=[
            pltpu.VMEM((b_per_w,), jnp.int32),
            pltpu.VMEM((b_per_w, CODE_DIM), jnp.float32),
            pltpu.SemaphoreType.DMA,
        ],
        compiler_params=pltpu.CompilerParams(use_tc_tiling_on_sc=False),
    )
    def gather_k(table_hbm, idx_hbm, out_hbm, idx_v, rows_v, sem):
        wid = lax.axis_index("s") * NC + lax.axis_index("c")
        base = wid * b_per_w
        pltpu.sync_copy(idx_hbm.at[pl.ds(base, b_per_w)], idx_v)
        pltpu.async_copy(table_hbm.at[idx_v], rows_v, sem).wait()
        pltpu.sync_copy(rows_v, out_hbm.at[pl.ds(base, b_per_w)])

    return gather_k


def kernel(z_e_x, codebook):
    flat = z_e_x.reshape(-1, CODE_DIM)
    n = flat.shape[0]
    half = n // 2
    idx_a = _vq_argmin(flat[:half], codebook)
    idx_b = _vq_argmin(flat[half:], codebook)
    idx = jnp.concatenate([idx_a, idx_b])
    out = _make_sc_gather(n)(codebook, idx)
    codes = out.reshape(z_e_x.shape)
    return (codes, codes)


# speedup vs baseline: 1.0580x; 1.0580x over previous
import functools

import jax
import jax.numpy as jnp
from jax import lax
from jax.experimental import pallas as pl
from jax.experimental.pallas import tpu as pltpu
from jax.experimental.pallas import tpu_sc as plsc

K_CODES = 1024
CODE_DIM = 64
ROW_BLOCK = 2048
NC = 2
NS = 16
NW = NC * NS


def _argmin_body(x_ref, cb_ref, idx_ref):
    x = x_ref[...]
    cb = cb_ref[...]
    xs = x * (-2.0)
    s2 = jax.lax.dot_general(xs, cb, (((1,), (1,)), ((), ())))
    x2 = jnp.sum(x * x, axis=1, keepdims=True)
    c2 = jnp.sum(cb * cb, axis=1)
    d = (x2 + c2[None, :]) + s2
    dmin = jnp.min(d, axis=1, keepdims=True)
    iota_k = jax.lax.broadcasted_iota(jnp.int32, (ROW_BLOCK, K_CODES), 1)
    idx_ref[...] = jnp.min(jnp.where(d == dmin, iota_k, K_CODES), axis=1)


def _vq_argmin(flat, codebook):
    n = flat.shape[0]
    return pl.pallas_call(
        _argmin_body,
        grid=(n // ROW_BLOCK,),
        in_specs=[
            pl.BlockSpec((ROW_BLOCK, CODE_DIM), lambda i: (i, 0)),
            pl.BlockSpec((K_CODES, CODE_DIM), lambda i: (0, 0)),
        ],
        out_specs=pl.BlockSpec((ROW_BLOCK,), lambda i: (i,)),
        out_shape=jax.ShapeDtypeStruct((n,), jnp.int32),
    )(flat, codebook)


def _make_sc_gather(n):
    b_per_w = n // NW
    mesh = plsc.VectorSubcoreMesh(core_axis_name="c", subcore_axis_name="s")

    @functools.partial(
        pl.kernel,
        mesh=mesh,
        out_type=jax.ShapeDtypeStruct((n, CODE_DIM), jnp.float32),
        scratch_types=[
            pltpu.VMEM((b_per_w,), jnp.int32),
            pltpu.VMEM((b_per_w, CODE_DIM), jnp.float32),
            pltpu.SemaphoreType.DMA,
        ],
        compiler_params=pltpu.CompilerParams(use_tc_tiling_on_sc=False),
    )
    def gather_k(table_hbm, idx_hbm, out_hbm, idx_v, rows_v, sem):
        wid = lax.axis_index("s") * NC + lax.axis_index("c")
        base = wid * b_per_w
        pltpu.sync_copy(idx_hbm.at[pl.ds(base, b_per_w)], idx_v)
        pltpu.async_copy(table_hbm.at[idx_v], rows_v, sem).wait()
        pltpu.sync_copy(rows_v, out_hbm.at[pl.ds(base, b_per_w)])

    return gather_k


def kernel(z_e_x, codebook):
    flat = z_e_x.reshape(-1, CODE_DIM)
    idx = _vq_argmin(flat, codebook)
    out = _make_sc_gather(flat.shape[0])(codebook, idx)
    codes = out.reshape(z_e_x.shape)
    return (codes, codes)


# speedup vs baseline: 1.0809x; 1.0216x over previous
import functools

import jax
import jax.numpy as jnp
from jax import lax
from jax.experimental import pallas as pl
from jax.experimental.pallas import tpu as pltpu
from jax.experimental.pallas import tpu_sc as plsc

K_CODES = 1024
CODE_DIM = 64
ROW_BLOCK = 4096
NC = 2
NS = 16
NW = NC * NS


def _argmin_body(x_ref, cb_ref, idx_ref):
    x = x_ref[...]
    cb = cb_ref[...]
    xs = x * (-2.0)
    s2 = jax.lax.dot_general(xs, cb, (((1,), (1,)), ((), ())))
    x2 = jnp.sum(x * x, axis=1, keepdims=True)
    c2 = jnp.sum(cb * cb, axis=1)
    d = (x2 + c2[None, :]) + s2
    dmin = jnp.min(d, axis=1, keepdims=True)
    iota_k = jax.lax.broadcasted_iota(jnp.int32, (ROW_BLOCK, K_CODES), 1)
    idx_ref[...] = jnp.min(jnp.where(d == dmin, iota_k, K_CODES), axis=1)


def _vq_argmin(flat, codebook):
    n = flat.shape[0]
    return pl.pallas_call(
        _argmin_body,
        grid=(n // ROW_BLOCK,),
        in_specs=[
            pl.BlockSpec((ROW_BLOCK, CODE_DIM), lambda i: (i, 0)),
            pl.BlockSpec((K_CODES, CODE_DIM), lambda i: (0, 0)),
        ],
        out_specs=pl.BlockSpec((ROW_BLOCK,), lambda i: (i,)),
        out_shape=jax.ShapeDtypeStruct((n,), jnp.int32),
    )(flat, codebook)


def _make_sc_gather(n):
    b_per_w = n // NW
    mesh = plsc.VectorSubcoreMesh(core_axis_name="c", subcore_axis_name="s")

    @functools.partial(
        pl.kernel,
        mesh=mesh,
        out_type=jax.ShapeDtypeStruct((n, CODE_DIM), jnp.float32),
        scratch_types=[
            pltpu.VMEM((b_per_w,), jnp.int32),
            pltpu.VMEM((b_per_w, CODE_DIM), jnp.float32),
            pltpu.SemaphoreType.DMA,
        ],
        compiler_params=pltpu.CompilerParams(use_tc_tiling_on_sc=False),
    )
    def gather_k(table_hbm, idx_hbm, out_hbm, idx_v, rows_v, sem):
        wid = lax.axis_index("s") * NC + lax.axis_index("c")
        base = wid * b_per_w
        pltpu.sync_copy(idx_hbm.at[pl.ds(base, b_per_w)], idx_v)
        pltpu.async_copy(table_hbm.at[idx_v], rows_v, sem).wait()
        pltpu.sync_copy(rows_v, out_hbm.at[pl.ds(base, b_per_w)])

    return gather_k


def kernel(z_e_x, codebook):
    flat = z_e_x.reshape(-1, CODE_DIM)
    idx = _vq_argmin(flat, codebook)
    out = _make_sc_gather(flat.shape[0])(codebook, idx)
    codes = out.reshape(z_e_x.shape)
    return (codes, codes)
